# Initial kernel scaffold; baseline (speedup 1.0000x reference)
import jax
import jax.numpy as jnp
from jax.experimental import pallas as pl


def kernel(self_etab, etab, E_idx, sequence, x_mask):
    raise NotImplementedError("write your pallas kernel here")


# speedup vs baseline: 4.5428x; 4.5428x over previous
import functools

import jax
import jax.numpy as jnp
from jax import lax
from jax.experimental import pallas as pl
from jax.experimental.pallas import tpu as pltpu
from jax.experimental.pallas import tpu_sc as plsc


def _sc_neighbor_labels(sequence, e_idx):
    B, L = sequence.shape
    K = e_idx.shape[2]
    A = 20
    info = plsc.get_sparse_core_info()
    NC, NS = info.num_cores, info.num_subcores
    NW = NC * NS
    N = B * L * K
    CH = N // NW
    assert N % NW == 0 and CH % 16 == 0 and (L * K) % CH == 0

    eflat = e_idx.reshape(NW, CH)
    per_b = (L * K) // CH

    mesh = plsc.VectorSubcoreMesh(core_axis_name="c", subcore_axis_name="s")

    @functools.partial(
        pl.kernel,
        out_type=jax.ShapeDtypeStruct((NW, CH), jnp.int32),
        mesh=mesh,
        compiler_params=pltpu.CompilerParams(needs_layout_passes=False),
        scratch_types=[
            pltpu.VMEM((L,), jnp.int32),
            pltpu.VMEM((CH,), jnp.int32),
            pltpu.VMEM((CH,), jnp.int32),
        ],
    )
    def k(seq_hbm, eidx_hbm, out_hbm, seq_v, idx_v, out_v):
        c = lax.axis_index("c")
        s = lax.axis_index("s")
        wid = s * NC + c
        pltpu.sync_copy(seq_hbm.at[wid // per_b], seq_v)
        pltpu.sync_copy(eidx_hbm.at[wid], idx_v)
        base = wid * CH

        def body(i, carry):
            idx = idx_v[pl.ds(i * 16, 16)]
            vals = plsc.load_gather(seq_v, [idx])
            pos = base + i * 16 + lax.iota(jnp.int32, 16)
            vals = jnp.where(lax.rem(pos, K) == 0, A, vals)
            out_v[pl.ds(i * 16, 16)] = vals
            return carry

        lax.fori_loop(0, CH // 16, body, 0)
        pltpu.sync_copy(out_v, out_hbm.at[wid])

    return k(sequence, eflat).reshape(B, L, K)


def _tc_nlpl(self_etab, etab, e_aa, sequence, x_mask):
    B, L, K, AA = etab.shape
    A = self_etab.shape[-1]
    TL = 128
    NT = L // TL

    def body(et_ref, ea_ref, se_ref, sq_ref, xm_ref, o1_ref, o2_ref):
        t = pl.program_id(1)
        et = et_ref[0]
        ea = ea_ref[0]
        sq = sq_ref[0, 0]
        xm = xm_ref[0, 0]
        lane = lax.broadcasted_iota(jnp.int32, (TL, K, AA), 2)
        mask = lax.rem(lane, A) == ea[:, :, None]
        masked = jnp.where(mask, et, 0.0).reshape(TL * K, AA)
        S = (
            lax.broadcasted_iota(jnp.int32, (AA, A), 0) // A
            == lax.broadcasted_iota(jnp.int32, (AA, A), 1)
        ).astype(jnp.float32)
        pair20 = jnp.dot(masked, S, preferred_element_type=jnp.float32)
        pair = jnp.sum(pair20.reshape(TL, K, A), axis=1)
        neg = -(se_ref[0] + pair)
        mx = jnp.max(neg, axis=-1, keepdims=True)
        lse = jnp.log(jnp.sum(jnp.exp(neg - mx), axis=-1)) + mx[:, 0]
        sel_mask = lax.broadcasted_iota(jnp.int32, (TL, A), 1) == sq[:, None]
        sel = jnp.sum(jnp.where(sel_mask, neg, 0.0), axis=-1)
        ps = jnp.sum((sel - lse) * xm).reshape(1, 1, 1)
        pm = jnp.sum(xm).reshape(1, 1, 1)
        z = jnp.zeros((1, 1, 1), jnp.float32)
        o1_ref[...] = jnp.where(t == 0, z, o1_ref[...]) + ps
        o2_ref[...] = jnp.where(t == 0, z, o2_ref[...]) + pm

    o1, o2 = pl.pallas_call(
        body,
        grid=(B, NT),
        in_specs=[
            pl.BlockSpec((1, TL, K, AA), lambda b, t: (b, t, 0, 0)),
            pl.BlockSpec((1, TL, K), lambda b, t: (b, t, 0)),
            pl.BlockSpec((1, TL, A), lambda b, t: (b, t, 0)),
            pl.BlockSpec((1, 1, TL), lambda b, t: (b * NT + t, 0, 0)),
            pl.BlockSpec((1, 1, TL), lambda b, t: (b * NT + t, 0, 0)),
        ],
        out_specs=[
            pl.BlockSpec((1, 1, 1), lambda b, t: (b, 0, 0)),
            pl.BlockSpec((1, 1, 1), lambda b, t: (b, 0, 0)),
        ],
        out_shape=[
            jax.ShapeDtypeStruct((B, 1, 1), jnp.float32),
            jax.ShapeDtypeStruct((B, 1, 1), jnp.float32),
        ],
    )(
        etab,
        e_aa,
        self_etab,
        sequence.reshape(B * NT, 1, TL),
        x_mask.reshape(B * NT, 1, TL),
    )
    return o1[:, 0, 0], o2[:, 0, 0]


def kernel(self_etab, etab, E_idx, sequence, x_mask):
    e_aa = _sc_neighbor_labels(sequence.astype(jnp.int32), E_idx.astype(jnp.int32))
    ps, pm = _tc_nlpl(self_etab, etab, e_aa, sequence.astype(jnp.int32), x_mask)
    return -jnp.mean(ps / pm)


# speedup vs baseline: 21.9841x; 4.8394x over previous
import functools

import jax
import jax.numpy as jnp
from jax import lax
from jax.experimental import pallas as pl
from jax.experimental.pallas import tpu as pltpu
from jax.experimental.pallas import tpu_sc as plsc

_A = 20


def _sc_neighbor_labels(sequence, e_idx_t):
    K, B, L = e_idx_t.shape
    info = plsc.get_sparse_core_info()
    NW = info.num_cores * info.num_subcores
    per_b = NW // B
    CL = L // per_b
    assert CL % 16 == 0

    mesh = plsc.VectorSubcoreMesh(core_axis_name="c", subcore_axis_name="s")

    @functools.partial(
        pl.kernel,
        out_type=jax.ShapeDtypeStruct((B, K, L), jnp.int32),
        mesh=mesh,
        compiler_params=pltpu.CompilerParams(needs_layout_passes=False),
        scratch_types=[
            pltpu.VMEM((L,), jnp.int32),
            pltpu.VMEM((K, CL), jnp.int32),
            pltpu.VMEM((K, CL), jnp.int32),
        ],
    )
    def k(seq_hbm, eidx_hbm, out_hbm, seq_v, idx_v, out_v):
        c = lax.axis_index("c")
        s = lax.axis_index("s")
        wid = s * info.num_cores + c
        b = wid // per_b
        l0 = (wid % per_b) * CL
        pltpu.sync_copy(seq_hbm.at[b], seq_v)
        pltpu.sync_copy(eidx_hbm.at[:, b, pl.ds(l0, CL)], idx_v)

        def row0(j, carry):
            out_v[0, pl.ds(j * 16, 16)] = jnp.full((16,), _A, jnp.int32)
            return carry

        lax.fori_loop(0, CL // 16, row0, 0)

        def body(i, carry):
            kk = 1 + i // (CL // 16)
            off = (i % (CL // 16)) * 16
            idx = idx_v[kk, pl.ds(off, 16)]
            out_v[kk, pl.ds(off, 16)] = plsc.load_gather(seq_v, [idx])
            return carry

        lax.fori_loop(0, (K - 1) * (CL // 16), body, 0)
        pltpu.sync_copy(out_v, out_hbm.at[b, :, pl.ds(l0, CL)])

    return k(sequence, e_idx_t)


def _tc_nlpl(self2, etab_t, e_aa_t, seq3, xm3):
    B, K, AA, L = etab_t.shape
    A = _A
    TL = 128
    NT = L // TL

    def body(et_ref, ea_ref, se_ref, sq_ref, xm_ref, o1_ref, o2_ref):
        t = pl.program_id(1)
        et = et_ref[0]
        ea = ea_ref[0]
        sq = sq_ref[0, 0]
        xm = xm_ref[0, 0]
        m_iota = lax.broadcasted_iota(jnp.int32, (K, AA, TL), 1)
        mask = lax.rem(m_iota, A) == ea[:, None, :]
        msum = jnp.sum(jnp.where(mask, et, 0.0), axis=0)
        S2 = (
            lax.broadcasted_iota(jnp.int32, (A, AA), 1) // A
            == lax.broadcasted_iota(jnp.int32, (A, AA), 0)
        ).astype(jnp.float32)
        pair = jnp.dot(S2, msum, preferred_element_type=jnp.float32)
        neg = -(se_ref[...] + pair)
        mx = jnp.max(neg, axis=0)
        lse = jnp.log(jnp.sum(jnp.exp(neg - mx[None, :]), axis=0)) + mx
        sel_mask = lax.broadcasted_iota(jnp.int32, (A, TL), 0) == sq[None, :]
        sel = jnp.sum(jnp.where(sel_mask, neg, 0.0), axis=0)
        ps = jnp.sum((sel - lse) * xm).reshape(1, 1, 1)
        pm = jnp.sum(xm).reshape(1, 1, 1)
        z = jnp.zeros((1, 1, 1), jnp.float32)
        o1_ref[...] = jnp.where(t == 0, z, o1_ref[...]) + ps
        o2_ref[...] = jnp.where(t == 0, z, o2_ref[...]) + pm

    o1, o2 = pl.pallas_call(
        body,
        grid=(B, NT),
        in_specs=[
            pl.BlockSpec((1, K, AA, TL), lambda b, t: (b, 0, 0, t)),
            pl.BlockSpec((1, K, TL), lambda b, t: (b, 0, t)),
            pl.BlockSpec((_A, TL), lambda b, t: (0, b * NT + t)),
            pl.BlockSpec((1, 1, TL), lambda b, t: (b * NT + t, 0, 0)),
            pl.BlockSpec((1, 1, TL), lambda b, t: (b * NT + t, 0, 0)),
        ],
        out_specs=[
            pl.BlockSpec((1, 1, 1), lambda b, t: (b, 0, 0)),
            pl.BlockSpec((1, 1, 1), lambda b, t: (b, 0, 0)),
        ],
        out_shape=[
            jax.ShapeDtypeStruct((B, 1, 1), jnp.float32),
            jax.ShapeDtypeStruct((B, 1, 1), jnp.float32),
        ],
    )(etab_t, e_aa_t, self2, seq3, xm3)
    return o1[:, 0, 0], o2[:, 0, 0]


def kernel(self_etab, etab, E_idx, sequence, x_mask):
    B, L, K, AA = etab.shape
    TL = 128
    NT = L // TL
    sequence = sequence.astype(jnp.int32)
    etab_t = jnp.transpose(etab, (0, 2, 3, 1))
    eidx_t = jnp.transpose(E_idx.astype(jnp.int32), (2, 0, 1))
    self2 = jnp.transpose(self_etab, (2, 0, 1)).reshape(_A, B * L)
    seq3 = sequence.reshape(B * NT, 1, TL)
    xm3 = x_mask.reshape(B * NT, 1, TL)
    e_aa_t = _sc_neighbor_labels(sequence, eidx_t)
    ps, pm = _tc_nlpl(self2, etab_t, e_aa_t, seq3, xm3)
    return -jnp.mean(ps / pm)


# speedup vs baseline: 24.4416x; 1.1118x over previous
import functools

import jax
import jax.numpy as jnp
from jax import lax
from jax.experimental import pallas as pl
from jax.experimental.pallas import tpu as pltpu
from jax.experimental.pallas import tpu_sc as plsc

_A = 20


def _sc_neighbor_labels(sequence, e_idx_t):
    K, B, L = e_idx_t.shape
    info = plsc.get_sparse_core_info()
    NW = info.num_cores * info.num_subcores
    per_b = NW // B
    CL = L // per_b
    assert CL % 16 == 0

    mesh = plsc.VectorSubcoreMesh(core_axis_name="c", subcore_axis_name="s")

    @functools.partial(
        pl.kernel,
        out_type=jax.ShapeDtypeStruct((B, K, L), jnp.int32),
        mesh=mesh,
        compiler_params=pltpu.CompilerParams(needs_layout_passes=False),
        scratch_types=[
            pltpu.VMEM((L,), jnp.int32),
            pltpu.VMEM((K, CL), jnp.int32),
            pltpu.VMEM((K, CL), jnp.int32),
        ],
    )
    def k(seq_hbm, eidx_hbm, out_hbm, seq_v, idx_v, out_v):
        c = lax.axis_index("c")
        s = lax.axis_index("s")
        wid = s * info.num_cores + c
        b = wid // per_b
        l0 = (wid % per_b) * CL
        pltpu.sync_copy(seq_hbm.at[b], seq_v)
        pltpu.sync_copy(eidx_hbm.at[:, b, pl.ds(l0, CL)], idx_v)

        def row0(j, carry):
            out_v[0, pl.ds(j * 16, 16)] = jnp.full((16,), _A, jnp.int32)
            return carry

        lax.fori_loop(0, CL // 16, row0, 0)

        def body(i, carry):
            kk = 1 + i // (CL // 16)
            off = (i % (CL // 16)) * 16
            idx = idx_v[kk, pl.ds(off, 16)]
            out_v[kk, pl.ds(off, 16)] = plsc.load_gather(seq_v, [idx])
            return carry

        lax.fori_loop(0, (K - 1) * (CL // 16), body, 0)
        pltpu.sync_copy(out_v, out_hbm.at[b, :, pl.ds(l0, CL)])

    return k(sequence, e_idx_t)


def _tc_nlpl(self2, etab_t, e_aa_t, seq3, xm3):
    B, K, AA, L = etab_t.shape
    A = _A
    TL = 256
    NT = L // TL

    def body(et_ref, ea_ref, se_ref, sq_ref, xm_ref, o1_ref, o2_ref):
        t = pl.program_id(1)
        et = et_ref[0]
        ea = ea_ref[0]
        sq = sq_ref[0, 0]
        xm = xm_ref[0, 0]
        m_iota = lax.broadcasted_iota(jnp.int32, (K, AA, TL), 1)
        mask = lax.rem(m_iota, A) == ea[:, None, :]
        msum = jnp.sum(jnp.where(mask, et, 0.0), axis=0)
        S2 = (
            lax.broadcasted_iota(jnp.int32, (A, AA), 1) // A
            == lax.broadcasted_iota(jnp.int32, (A, AA), 0)
        ).astype(jnp.float32)
        pair = jnp.dot(S2, msum, preferred_element_type=jnp.float32)
        neg = -(se_ref[...] + pair)
        mx = jnp.max(neg, axis=0)
        lse = jnp.log(jnp.sum(jnp.exp(neg - mx[None, :]), axis=0)) + mx
        sel_mask = lax.broadcasted_iota(jnp.int32, (A, TL), 0) == sq[None, :]
        sel = jnp.sum(jnp.where(sel_mask, neg, 0.0), axis=0)
        ps = jnp.sum((sel - lse) * xm).reshape(1, 1, 1)
        pm = jnp.sum(xm).reshape(1, 1, 1)
        z = jnp.zeros((1, 1, 1), jnp.float32)
        o1_ref[...] = jnp.where(t == 0, z, o1_ref[...]) + ps
        o2_ref[...] = jnp.where(t == 0, z, o2_ref[...]) + pm

    o1, o2 = pl.pallas_call(
        body,
        grid=(B, NT),
        in_specs=[
            pl.BlockSpec((1, K, AA, TL), lambda b, t: (b, 0, 0, t)),
            pl.BlockSpec((1, K, TL), lambda b, t: (b, 0, t)),
            pl.BlockSpec((_A, TL), lambda b, t: (0, b * NT + t)),
            pl.BlockSpec((1, 1, TL), lambda b, t: (b * NT + t, 0, 0)),
            pl.BlockSpec((1, 1, TL), lambda b, t: (b * NT + t, 0, 0)),
        ],
        out_specs=[
            pl.BlockSpec((1, 1, 1), lambda b, t: (b, 0, 0)),
            pl.BlockSpec((1, 1, 1), lambda b, t: (b, 0, 0)),
        ],
        out_shape=[
            jax.ShapeDtypeStruct((B, 1, 1), jnp.float32),
            jax.ShapeDtypeStruct((B, 1, 1), jnp.float32),
        ],
    )(etab_t, e_aa_t, self2, seq3, xm3)
    return o1[:, 0, 0], o2[:, 0, 0]


def kernel(self_etab, etab, E_idx, sequence, x_mask):
    B, L, K, AA = etab.shape
    TL = 256
    NT = L // TL
    sequence = sequence.astype(jnp.int32)
    etab_t = jnp.transpose(etab, (0, 2, 3, 1))
    eidx_t = jnp.transpose(E_idx.astype(jnp.int32), (2, 0, 1))
    self2 = jnp.transpose(self_etab, (2, 0, 1)).reshape(_A, B * L)
    seq3 = sequence.reshape(B * NT, 1, TL)
    xm3 = x_mask.reshape(B * NT, 1, TL)
    e_aa_t = _sc_neighbor_labels(sequence, eidx_t)
    ps, pm = _tc_nlpl(self2, etab_t, e_aa_t, seq3, xm3)
    return -jnp.mean(ps / pm)
